# baseline (speedup 1.0000x reference)
import functools

import jax
import jax.numpy as jnp
from jax.experimental import pallas as pl
from jax.experimental.pallas import tpu as pltpu
from jax.experimental.pallas import tpu_sc as plsc

BATCH = 16384
EMBED_DIM = 64
HIDDEN_DIM = 128
OUTPUT_DIM = 64

_NUM_CORES = 2
_NUM_SUBCORES = 16
_NUM_WORKERS = _NUM_CORES * _NUM_SUBCORES
_B_PER_WORKER = BATCH // _NUM_WORKERS


def _sc_gather(emb_table, idx):
    mesh = plsc.VectorSubcoreMesh(core_axis_name="c", subcore_axis_name="s")

    @functools.partial(
        pl.kernel,
        mesh=mesh,
        out_type=jax.ShapeDtypeStruct((BATCH, EMBED_DIM), emb_table.dtype),
        scratch_types=[
            pltpu.VMEM((_B_PER_WORKER,), jnp.int32),
            pltpu.SemaphoreType.DMA,
        ],
    )
    def gather_kernel(table_hbm, idx_hbm, out_hbm, idx_v, sem):
        wid = jax.lax.axis_index("s") * _NUM_CORES + jax.lax.axis_index("c")
        base = wid * _B_PER_WORKER
        pltpu.sync_copy(idx_hbm.at[pl.ds(base, _B_PER_WORKER)], idx_v)

        @pl.loop(0, _B_PER_WORKER // 16)
        def _fire(g):
            vec = idx_v[pl.ds(g * 16, 16)]
            for j in range(16):
                row = vec[j]
                pltpu.make_async_copy(
                    table_hbm.at[pl.ds(row, 1)],
                    out_hbm.at[pl.ds(base + g * 16 + j, 1)],
                    sem,
                ).start()

        @pl.loop(0, _B_PER_WORKER)
        def _drain(i):
            pltpu.make_async_copy(
                table_hbm.at[pl.ds(0, 1)],
                out_hbm.at[pl.ds(base + i, 1)],
                sem,
            ).wait()

    return gather_kernel(emb_table, idx)


_MLP_BLOCK = 2048


def _mlp_kernel(x_ref, w1_ref, b1_ref, w2_ref, b2_ref, o_ref):
    h = jnp.dot(x_ref[...], w1_ref[...], preferred_element_type=jnp.float32)
    h = jnp.maximum(h + b1_ref[...], 0.0)
    o_ref[...] = (
        jnp.dot(h, w2_ref[...], preferred_element_type=jnp.float32) + b2_ref[...]
    )


def _tc_mlp(x, W1, b1, W2, b2):
    grid = (BATCH // _MLP_BLOCK,)
    return pl.pallas_call(
        _mlp_kernel,
        grid=grid,
        in_specs=[
            pl.BlockSpec((_MLP_BLOCK, EMBED_DIM), lambda i: (i, 0)),
            pl.BlockSpec((EMBED_DIM, HIDDEN_DIM), lambda i: (0, 0)),
            pl.BlockSpec((1, HIDDEN_DIM), lambda i: (0, 0)),
            pl.BlockSpec((HIDDEN_DIM, OUTPUT_DIM), lambda i: (0, 0)),
            pl.BlockSpec((1, OUTPUT_DIM), lambda i: (0, 0)),
        ],
        out_specs=pl.BlockSpec((_MLP_BLOCK, OUTPUT_DIM), lambda i: (i, 0)),
        out_shape=jax.ShapeDtypeStruct((BATCH, OUTPUT_DIM), jnp.float32),
    )(x, W1, b1.reshape(1, HIDDEN_DIM), W2, b2.reshape(1, OUTPUT_DIM))


def kernel(llm_ids, emb_table, W1, b1, W2, b2):
    ids = llm_ids.astype(jnp.int32)
    embedded = _sc_gather(emb_table, ids)
    return _tc_mlp(embedded, W1, b1, W2, b2)


# speedup vs baseline: 2.7929x; 2.7929x over previous
import functools

import jax
import jax.numpy as jnp
from jax.experimental import pallas as pl
from jax.experimental.pallas import tpu as pltpu
from jax.experimental.pallas import tpu_sc as plsc

BATCH = 16384
EMBED_DIM = 64
HIDDEN_DIM = 128
OUTPUT_DIM = 64
NUM_ROWS = 100000

_NUM_CORES = 2
_NUM_SUBCORES = 16
_NUM_WORKERS = _NUM_CORES * _NUM_SUBCORES
_B_PER_WORKER = BATCH // _NUM_WORKERS
_WIDE = 2 * EMBED_DIM

_HALF_ROWS = NUM_ROWS // 2
_RELAYOUT_BLOCK = 2000


def _relayout_kernel(top_ref, bot_ref, o_ref):
    o_ref[:, :EMBED_DIM] = top_ref[...]
    o_ref[:, EMBED_DIM:] = bot_ref[...]


def _tc_relayout(emb_table):
    grid = (_HALF_ROWS // _RELAYOUT_BLOCK,)
    nblk = _HALF_ROWS // _RELAYOUT_BLOCK
    return pl.pallas_call(
        _relayout_kernel,
        grid=grid,
        in_specs=[
            pl.BlockSpec((_RELAYOUT_BLOCK, EMBED_DIM), lambda i: (i, 0)),
            pl.BlockSpec((_RELAYOUT_BLOCK, EMBED_DIM), lambda i: (i + nblk, 0)),
        ],
        out_specs=pl.BlockSpec((_RELAYOUT_BLOCK, _WIDE), lambda i: (i, 0)),
        out_shape=jax.ShapeDtypeStruct((_HALF_ROWS, _WIDE), jnp.float32),
    )(emb_table, emb_table)


def _sc_gather_wide(table_wide, idx_half):
    mesh = plsc.VectorSubcoreMesh(core_axis_name="c", subcore_axis_name="s")

    @functools.partial(
        pl.kernel,
        mesh=mesh,
        out_type=jax.ShapeDtypeStruct((BATCH, _WIDE), table_wide.dtype),
        scratch_types=[
            pltpu.VMEM((_B_PER_WORKER,), jnp.int32),
            pltpu.VMEM((_B_PER_WORKER, _WIDE), jnp.float32),
            pltpu.SemaphoreType.DMA,
        ],
    )
    def gather_kernel(table_hbm, idx_hbm, out_hbm, idx_v, rows_v, sem):
        wid = jax.lax.axis_index("s") * _NUM_CORES + jax.lax.axis_index("c")
        base = wid * _B_PER_WORKER
        pltpu.sync_copy(idx_hbm.at[pl.ds(base, _B_PER_WORKER)], idx_v)
        pltpu.async_copy(table_hbm.at[idx_v], rows_v, sem).wait()
        pltpu.sync_copy(rows_v, out_hbm.at[pl.ds(base, _B_PER_WORKER)])

    return gather_kernel(table_wide, idx_half)


_MLP_BLOCK = 2048


def _mlp_kernel(x_ref, par_ref, w1_ref, b1_ref, w2_ref, b2_ref, o_ref):
    x = x_ref[...]
    odd = par_ref[...] > 0
    emb = jnp.where(odd, x[:, EMBED_DIM:], x[:, :EMBED_DIM])
    h = jnp.dot(emb, w1_ref[...], preferred_element_type=jnp.float32)
    h = jnp.maximum(h + b1_ref[...], 0.0)
    o_ref[...] = (
        jnp.dot(h, w2_ref[...], preferred_element_type=jnp.float32) + b2_ref[...]
    )


def _tc_mlp(x_wide, parity, W1, b1, W2, b2):
    grid = (BATCH // _MLP_BLOCK,)
    return pl.pallas_call(
        _mlp_kernel,
        grid=grid,
        in_specs=[
            pl.BlockSpec((_MLP_BLOCK, _WIDE), lambda i: (i, 0)),
            pl.BlockSpec((_MLP_BLOCK, 1), lambda i: (i, 0)),
            pl.BlockSpec((EMBED_DIM, HIDDEN_DIM), lambda i: (0, 0)),
            pl.BlockSpec((1, HIDDEN_DIM), lambda i: (0, 0)),
            pl.BlockSpec((HIDDEN_DIM, OUTPUT_DIM), lambda i: (0, 0)),
            pl.BlockSpec((1, OUTPUT_DIM), lambda i: (0, 0)),
        ],
        out_specs=pl.BlockSpec((_MLP_BLOCK, OUTPUT_DIM), lambda i: (i, 0)),
        out_shape=jax.ShapeDtypeStruct((BATCH, OUTPUT_DIM), jnp.float32),
    )(
        x_wide,
        parity,
        W1,
        b1.reshape(1, HIDDEN_DIM),
        W2,
        b2.reshape(1, OUTPUT_DIM),
    )


def kernel(llm_ids, emb_table, W1, b1, W2, b2):
    ids = llm_ids.astype(jnp.int32)
    table_wide = _tc_relayout(emb_table)
    in_bottom = ids >= _HALF_ROWS
    wide = _sc_gather_wide(table_wide, ids - _HALF_ROWS * in_bottom)
    selector = in_bottom.astype(jnp.int32).reshape(BATCH, 1)
    return _tc_mlp(wide, selector, W1, b1, W2, b2)


# speedup vs baseline: 3.5417x; 1.2681x over previous
import functools

import jax
import jax.numpy as jnp
from jax.experimental import pallas as pl
from jax.experimental.pallas import tpu as pltpu
from jax.experimental.pallas import tpu_sc as plsc

BATCH = 16384
EMBED_DIM = 64
HIDDEN_DIM = 128
OUTPUT_DIM = 64
NUM_ROWS = 100000

_NUM_CORES = 2
_NUM_SUBCORES = 16
_NUM_WORKERS = _NUM_CORES * _NUM_SUBCORES
_B_PER_WORKER = BATCH // _NUM_WORKERS
_WIDE = 2 * EMBED_DIM

_RELAYOUT_BLOCK = 1024
_NBLK = 49
_SPLIT = _RELAYOUT_BLOCK * _NBLK


def _relayout_kernel(top_ref, bot_ref, o_ref):
    o_ref[:, :EMBED_DIM] = top_ref[...].T
    o_ref[:, EMBED_DIM:] = bot_ref[...].T


def _tc_relayout(table_t):
    grid = (_NBLK,)
    return pl.pallas_call(
        _relayout_kernel,
        grid=grid,
        in_specs=[
            pl.BlockSpec((EMBED_DIM, _RELAYOUT_BLOCK), lambda i: (0, i)),
            pl.BlockSpec((EMBED_DIM, _RELAYOUT_BLOCK), lambda i: (0, i + _NBLK)),
        ],
        out_specs=pl.BlockSpec((_RELAYOUT_BLOCK, _WIDE), lambda i: (i, 0)),
        out_shape=jax.ShapeDtypeStruct((_SPLIT, _WIDE), jnp.float32),
    )(table_t, table_t)


def _sc_gather_wide(table_wide, idx_half):
    mesh = plsc.VectorSubcoreMesh(core_axis_name="c", subcore_axis_name="s")

    @functools.partial(
        pl.kernel,
        mesh=mesh,
        out_type=jax.ShapeDtypeStruct((BATCH, _WIDE), table_wide.dtype),
        scratch_types=[
            pltpu.VMEM((_B_PER_WORKER,), jnp.int32),
            pltpu.VMEM((_B_PER_WORKER, _WIDE), jnp.float32),
            pltpu.SemaphoreType.DMA,
        ],
    )
    def gather_kernel(table_hbm, idx_hbm, out_hbm, idx_v, rows_v, sem):
        wid = jax.lax.axis_index("s") * _NUM_CORES + jax.lax.axis_index("c")
        base = wid * _B_PER_WORKER
        pltpu.sync_copy(idx_hbm.at[pl.ds(base, _B_PER_WORKER)], idx_v)
        pltpu.async_copy(table_hbm.at[idx_v], rows_v, sem).wait()
        pltpu.sync_copy(rows_v, out_hbm.at[pl.ds(base, _B_PER_WORKER)])

    return gather_kernel(table_wide, idx_half)


_MLP_BLOCK = 4096


def _mlp_kernel(x_ref, par_ref, w1_ref, b1_ref, w2_ref, b2_ref, o_ref):
    x = x_ref[...]
    odd = par_ref[...] > 0
    emb = jnp.where(odd, x[:, EMBED_DIM:], x[:, :EMBED_DIM])
    h = jnp.dot(emb, w1_ref[...], preferred_element_type=jnp.float32)
    h = jnp.maximum(h + b1_ref[...], 0.0)
    o_ref[...] = (
        jnp.dot(h, w2_ref[...], preferred_element_type=jnp.float32) + b2_ref[...]
    )


def _tc_mlp(x_wide, parity, W1, b1, W2, b2):
    grid = (BATCH // _MLP_BLOCK,)
    return pl.pallas_call(
        _mlp_kernel,
        grid=grid,
        in_specs=[
            pl.BlockSpec((_MLP_BLOCK, _WIDE), lambda i: (i, 0)),
            pl.BlockSpec((_MLP_BLOCK, 1), lambda i: (i, 0)),
            pl.BlockSpec((EMBED_DIM, HIDDEN_DIM), lambda i: (0, 0)),
            pl.BlockSpec((1, HIDDEN_DIM), lambda i: (0, 0)),
            pl.BlockSpec((HIDDEN_DIM, OUTPUT_DIM), lambda i: (0, 0)),
            pl.BlockSpec((1, OUTPUT_DIM), lambda i: (0, 0)),
        ],
        out_specs=pl.BlockSpec((_MLP_BLOCK, OUTPUT_DIM), lambda i: (i, 0)),
        out_shape=jax.ShapeDtypeStruct((BATCH, OUTPUT_DIM), jnp.float32),
    )(
        x_wide,
        parity,
        W1,
        b1.reshape(1, HIDDEN_DIM),
        W2,
        b2.reshape(1, OUTPUT_DIM),
    )


def kernel(llm_ids, emb_table, W1, b1, W2, b2):
    ids = llm_ids.astype(jnp.int32)
    table_wide = _tc_relayout(emb_table.T)
    in_bottom = ids >= _SPLIT
    wide = _sc_gather_wide(table_wide, ids - _SPLIT * in_bottom)
    selector = in_bottom.astype(jnp.int32).reshape(BATCH, 1)
    return _tc_mlp(wide, selector, W1, b1, W2, b2)


# speedup vs baseline: 3.8466x; 1.0861x over previous
import functools

import jax
import jax.numpy as jnp
from jax.experimental import pallas as pl
from jax.experimental.pallas import tpu as pltpu
from jax.experimental.pallas import tpu_sc as plsc

BATCH = 16384
EMBED_DIM = 64
HIDDEN_DIM = 128
OUTPUT_DIM = 64
NUM_ROWS = 100000

_NUM_CORES = 2
_NUM_SUBCORES = 16
_NUM_WORKERS = _NUM_CORES * _NUM_SUBCORES
_B_PER_WORKER = BATCH // _NUM_WORKERS
_WIDE = 2 * EMBED_DIM

_RELAYOUT_BLOCK = 1024
_NBLK = 49
_SPLIT = _RELAYOUT_BLOCK * _NBLK


def _relayout_kernel(top_ref, bot_ref, o_ref):
    o_ref[:, :EMBED_DIM] = top_ref[...].T
    o_ref[:, EMBED_DIM:] = bot_ref[...].T


def _tc_relayout(table_t):
    grid = (_NBLK,)
    return pl.pallas_call(
        _relayout_kernel,
        grid=grid,
        in_specs=[
            pl.BlockSpec((EMBED_DIM, _RELAYOUT_BLOCK), lambda i: (0, i)),
            pl.BlockSpec((EMBED_DIM, _RELAYOUT_BLOCK), lambda i: (0, i + _NBLK)),
        ],
        out_specs=pl.BlockSpec((_RELAYOUT_BLOCK, _WIDE), lambda i: (i, 0)),
        out_shape=jax.ShapeDtypeStruct((_SPLIT, _WIDE), jnp.float32),
        compiler_params=pltpu.CompilerParams(
            dimension_semantics=("parallel",)
        ),
    )(table_t, table_t)


def _sc_gather_wide(table_wide, idx_half):
    mesh = plsc.VectorSubcoreMesh(core_axis_name="c", subcore_axis_name="s")

    @functools.partial(
        pl.kernel,
        mesh=mesh,
        out_type=jax.ShapeDtypeStruct((BATCH, _WIDE), table_wide.dtype),
        scratch_types=[
            pltpu.VMEM((_B_PER_WORKER,), jnp.int32),
            pltpu.VMEM((_B_PER_WORKER, _WIDE), jnp.float32),
            pltpu.SemaphoreType.DMA,
        ],
    )
    def gather_kernel(table_hbm, idx_hbm, out_hbm, idx_v, rows_v, sem):
        wid = jax.lax.axis_index("s") * _NUM_CORES + jax.lax.axis_index("c")
        base = wid * _B_PER_WORKER
        pltpu.sync_copy(idx_hbm.at[pl.ds(base, _B_PER_WORKER)], idx_v)
        pltpu.async_copy(table_hbm.at[idx_v], rows_v, sem).wait()
        pltpu.sync_copy(rows_v, out_hbm.at[pl.ds(base, _B_PER_WORKER)])

    return gather_kernel(table_wide, idx_half)


_MLP_BLOCK = 4096


def _mlp_kernel(x_ref, par_ref, w1_ref, b1_ref, w2_ref, b2_ref, o_ref):
    x = x_ref[...]
    odd = par_ref[...] > 0
    emb = jnp.where(odd, x[:, EMBED_DIM:], x[:, :EMBED_DIM])
    h = jnp.dot(emb, w1_ref[...], preferred_element_type=jnp.float32)
    h = jnp.maximum(h + b1_ref[...], 0.0)
    o_ref[...] = (
        jax.lax.dot_general(
            w2_ref[...],
            h,
            (((0,), (1,)), ((), ())),
            preferred_element_type=jnp.float32,
        )
        + b2_ref[...]
    )


def _tc_mlp(x_wide, parity, W1, b1, W2, b2):
    grid = (BATCH // _MLP_BLOCK,)
    out_t = pl.pallas_call(
        _mlp_kernel,
        grid=grid,
        in_specs=[
            pl.BlockSpec((_MLP_BLOCK, _WIDE), lambda i: (i, 0)),
            pl.BlockSpec((_MLP_BLOCK, 1), lambda i: (i, 0)),
            pl.BlockSpec((EMBED_DIM, HIDDEN_DIM), lambda i: (0, 0)),
            pl.BlockSpec((1, HIDDEN_DIM), lambda i: (0, 0)),
            pl.BlockSpec((HIDDEN_DIM, OUTPUT_DIM), lambda i: (0, 0)),
            pl.BlockSpec((OUTPUT_DIM, 1), lambda i: (0, 0)),
        ],
        out_specs=pl.BlockSpec((OUTPUT_DIM, _MLP_BLOCK), lambda i: (0, i)),
        out_shape=jax.ShapeDtypeStruct((OUTPUT_DIM, BATCH), jnp.float32),
        compiler_params=pltpu.CompilerParams(
            dimension_semantics=("parallel",)
        ),
    )(
        x_wide,
        parity,
        W1,
        b1.reshape(1, HIDDEN_DIM),
        W2,
        b2.reshape(OUTPUT_DIM, 1),
    )
    return out_t.T


def kernel(llm_ids, emb_table, W1, b1, W2, b2):
    ids = llm_ids.astype(jnp.int32)
    table_wide = _tc_relayout(emb_table.T)
    in_bottom = ids >= _SPLIT
    wide = _sc_gather_wide(table_wide, ids - _SPLIT * in_bottom)
    selector = in_bottom.astype(jnp.int32).reshape(BATCH, 1)
    return _tc_mlp(wide, selector, W1, b1, W2, b2)


# speedup vs baseline: 4.7824x; 1.2433x over previous
import functools

import jax
import jax.numpy as jnp
from jax.experimental import pallas as pl
from jax.experimental.pallas import tpu as pltpu
from jax.experimental.pallas import tpu_sc as plsc

BATCH = 16384
EMBED_DIM = 64
HIDDEN_DIM = 128
OUTPUT_DIM = 64
NUM_ROWS = 100000

_NUM_CORES = 2
_NUM_SUBCORES = 16
_NUM_WORKERS = _NUM_CORES * _NUM_SUBCORES
_B_PER_WORKER = BATCH // _NUM_WORKERS
_WIDE = 2 * EMBED_DIM

_RELAYOUT_BLOCK = 4096
_NBLK = 13
_SPLIT = _RELAYOUT_BLOCK * _NBLK
_LAST_IN_BLK = (NUM_ROWS - 1) // _RELAYOUT_BLOCK


def _relayout_kernel(top_ref, bot_ref, o_ref):
    o_ref[:, :EMBED_DIM] = top_ref[...].T
    o_ref[:, EMBED_DIM:] = bot_ref[...].T


def _tc_relayout(table_t):
    grid = (_NBLK,)
    return pl.pallas_call(
        _relayout_kernel,
        grid=grid,
        in_specs=[
            pl.BlockSpec((EMBED_DIM, _RELAYOUT_BLOCK), lambda i: (0, i)),
            pl.BlockSpec(
                (EMBED_DIM, _RELAYOUT_BLOCK),
                lambda i: (0, jnp.minimum(i + _NBLK, _LAST_IN_BLK)),
            ),
        ],
        out_specs=pl.BlockSpec((_RELAYOUT_BLOCK, _WIDE), lambda i: (i, 0)),
        out_shape=jax.ShapeDtypeStruct((_SPLIT, _WIDE), jnp.float32),
        compiler_params=pltpu.CompilerParams(
            dimension_semantics=("parallel",)
        ),
    )(table_t, table_t)


def _sc_gather_wide(table_wide, idx_half):
    mesh = plsc.VectorSubcoreMesh(core_axis_name="c", subcore_axis_name="s")

    @functools.partial(
        pl.kernel,
        mesh=mesh,
        out_type=jax.ShapeDtypeStruct((BATCH, _WIDE), table_wide.dtype),
        scratch_types=[
            pltpu.VMEM((_B_PER_WORKER,), jnp.int32),
            pltpu.VMEM((_B_PER_WORKER, _WIDE), jnp.float32),
            pltpu.SemaphoreType.DMA,
        ],
    )
    def gather_kernel(table_hbm, idx_hbm, out_hbm, idx_v, rows_v, sem):
        wid = jax.lax.axis_index("s") * _NUM_CORES + jax.lax.axis_index("c")
        base = wid * _B_PER_WORKER
        pltpu.sync_copy(idx_hbm.at[pl.ds(base, _B_PER_WORKER)], idx_v)
        pltpu.async_copy(table_hbm.at[idx_v], rows_v, sem).wait()
        pltpu.sync_copy(rows_v, out_hbm.at[pl.ds(base, _B_PER_WORKER)])

    return gather_kernel(table_wide, idx_half)


_MLP_BLOCK = 8192


def _mlp_kernel(x_ref, par_ref, w1_ref, b1_ref, w2_ref, b2_ref, o_ref):
    x = x_ref[...]
    odd = par_ref[...] > 0
    emb = jnp.where(odd, x[:, EMBED_DIM:], x[:, :EMBED_DIM])
    h = jnp.dot(emb, w1_ref[...], preferred_element_type=jnp.float32)
    h = jnp.maximum(h + b1_ref[...], 0.0)
    o_ref[...] = (
        jax.lax.dot_general(
            w2_ref[...],
            h,
            (((0,), (1,)), ((), ())),
            preferred_element_type=jnp.float32,
        )
        + b2_ref[...]
    )


def _tc_mlp(x_wide, parity, W1, b1, W2, b2):
    grid = (BATCH // _MLP_BLOCK,)
    out_t = pl.pallas_call(
        _mlp_kernel,
        grid=grid,
        in_specs=[
            pl.BlockSpec((_MLP_BLOCK, _WIDE), lambda i: (i, 0)),
            pl.BlockSpec((_MLP_BLOCK, 1), lambda i: (i, 0)),
            pl.BlockSpec((EMBED_DIM, HIDDEN_DIM), lambda i: (0, 0)),
            pl.BlockSpec((1, HIDDEN_DIM), lambda i: (0, 0)),
            pl.BlockSpec((HIDDEN_DIM, OUTPUT_DIM), lambda i: (0, 0)),
            pl.BlockSpec((OUTPUT_DIM, 1), lambda i: (0, 0)),
        ],
        out_specs=pl.BlockSpec((OUTPUT_DIM, _MLP_BLOCK), lambda i: (0, i)),
        out_shape=jax.ShapeDtypeStruct((OUTPUT_DIM, BATCH), jnp.float32),
        compiler_params=pltpu.CompilerParams(
            dimension_semantics=("parallel",)
        ),
    )(
        x_wide,
        parity,
        W1,
        b1.reshape(1, HIDDEN_DIM),
        W2,
        b2.reshape(OUTPUT_DIM, 1),
    )
    return out_t.T


def kernel(llm_ids, emb_table, W1, b1, W2, b2):
    ids = llm_ids.astype(jnp.int32)
    table_wide = _tc_relayout(emb_table.T)
    in_bottom = ids >= _SPLIT
    wide = _sc_gather_wide(table_wide, ids - _SPLIT * in_bottom)
    selector = in_bottom.astype(jnp.int32).reshape(BATCH, 1)
    return _tc_mlp(wide, selector, W1, b1, W2, b2)
